# baseline (speedup 1.0000x reference)
import functools

import jax
import jax.numpy as jnp
from jax import lax
from jax.experimental import pallas as pl
from jax.experimental.pallas import tpu as pltpu
from jax.experimental.pallas import tpu_sc as plsc

_BIN = 512
_K = 16
_DMULT = 0.1
_WROW = 384
_WPACK = 128
_CHUNK = 128


def _block_body(parts_ref, packed_ref):
    b = pl.program_id(0)
    blk = parts_ref[0]
    x = blk[:, :256]
    g = blk[:, 256].astype(jnp.int32) - b * 8192
    na = jnp.sum(x * x, axis=1, keepdims=True)
    gram = lax.dot_general(x, x, (((1,), (1,)), ((), ())),
                           preferred_element_type=jnp.float32)
    dsq = na - 2.0 * gram + jnp.transpose(na)
    dm = jnp.exp(-_DMULT * jnp.sqrt(jnp.maximum(dsq, 1e-6)))

    colj = lax.broadcasted_iota(jnp.int32, (_BIN, _BIN), 1)
    comp_base = colj * 8192 + g[None, :]
    big = jnp.int32(2 ** 30)

    dsts, vals = [], []
    cur = dm
    for _ in range(_K):
        m = jnp.max(cur, axis=1, keepdims=True)
        comp = jnp.where(cur == m, comp_base, big)
        r = jnp.min(comp, axis=1)
        selj = r // 8192
        dsts.append(r - selj * 8192)
        vals.append(m[:, 0])
        cur = jnp.where(colj == selj[:, None], -1.0, cur)

    dst = jnp.stack(dsts, axis=1)
    val = jnp.stack(vals, axis=1)

    rank = jnp.sum((dst[:, None, :] < dst[:, :, None]).astype(jnp.int32),
                   axis=2)
    sd, sv = [], []
    for u in range(_K):
        sel = rank == u
        sd.append(jnp.sum(jnp.where(sel, dst, 0), axis=1))
        sv.append(jnp.sum(jnp.where(sel, val, 0.0), axis=1))
    packed_ref[0] = jnp.concatenate(
        [jnp.stack(sd, axis=1),
         lax.bitcast_convert_type(jnp.stack(sv, axis=1), jnp.int32),
         jnp.zeros((_BIN, _WPACK - 2 * _K), jnp.int32)], axis=1)


def _topk_blocks(parts, B, nbins):
    return pl.pallas_call(
        _block_body,
        grid=(B, nbins),
        in_specs=[pl.BlockSpec((1, _BIN, _WROW), lambda b, n: (b, n, 0))],
        out_specs=pl.BlockSpec((1, _BIN, _WPACK), lambda b, n: (b, n, 0)),
        out_shape=jax.ShapeDtypeStruct((B, nbins * _BIN, _WPACK), jnp.int32),
    )(parts)


def _make_row_scatter(M, W):
    info = plsc.get_sparse_core_info()
    NC, NS = info.num_cores, info.num_subcores
    per_w = M // (NC * NS)
    n_chunks = per_w // _CHUNK
    mesh = plsc.VectorSubcoreMesh(core_axis_name="c", subcore_axis_name="s")

    @functools.partial(
        pl.kernel, mesh=mesh,
        out_type=jax.ShapeDtypeStruct((M, W), jnp.float32),
        scratch_types=[pltpu.VMEM((_CHUNK,), jnp.int32),
                       pltpu.VMEM((_CHUNK, W), jnp.float32),
                       pltpu.SemaphoreType.DMA],
    )
    def sk(rows_hbm, idx_hbm, out_hbm, idx_v, rows_v, sem):
        wid = lax.axis_index("s") * NC + lax.axis_index("c")
        base = wid * per_w
        for c in range(n_chunks):
            off = base + c * _CHUNK
            pltpu.sync_copy(rows_hbm.at[pl.ds(off, _CHUNK)], rows_v)
            pltpu.sync_copy(idx_hbm.at[pl.ds(off, _CHUNK)], idx_v)
            pltpu.async_copy(rows_v, out_hbm.at[idx_v], sem).wait()

    return sk


def _make_row_gather(M, W):
    info = plsc.get_sparse_core_info()
    NC, NS = info.num_cores, info.num_subcores
    per_w = M // (NC * NS)
    n_chunks = per_w // _CHUNK
    mesh = plsc.VectorSubcoreMesh(core_axis_name="c", subcore_axis_name="s")

    @functools.partial(
        pl.kernel, mesh=mesh,
        out_type=jax.ShapeDtypeStruct((M, W), jnp.int32),
        scratch_types=[pltpu.VMEM((_CHUNK,), jnp.int32),
                       pltpu.VMEM((_CHUNK, W), jnp.int32),
                       pltpu.SemaphoreType.DMA],
    )
    def gk(table_hbm, idx_hbm, out_hbm, idx_v, rows_v, sem):
        wid = lax.axis_index("s") * NC + lax.axis_index("c")
        base = wid * per_w
        for c in range(n_chunks):
            off = base + c * _CHUNK
            pltpu.sync_copy(idx_hbm.at[pl.ds(off, _CHUNK)], idx_v)
            pltpu.async_copy(table_hbm.at[idx_v], rows_v, sem).wait()
            pltpu.sync_copy(rows_v, out_hbm.at[pl.ds(off, _CHUNK)])

    return gk


def kernel(inputs, codebook):
    B, N, D = inputs.shape
    nbins = N // _BIN
    M = B * N

    mul = inputs @ codebook[:, : nbins // 2]
    cmul = jnp.concatenate([mul, -mul], axis=-1)
    bin_idx = jnp.argmax(cmul, axis=-1).astype(jnp.int32)

    oh = (bin_idx[:, None, :] ==
          jnp.arange(nbins, dtype=jnp.int32)[None, :, None]).astype(jnp.int32)
    csum = jnp.cumsum(oh, axis=2)
    counts = csum[:, :, -1]
    starts = jnp.cumsum(counts, axis=1) - counts
    pos = (jnp.sum(oh * csum, axis=1) - 1
           + jnp.sum(oh * starts[:, :, None], axis=1))
    pos_g = (pos + jnp.arange(B, dtype=jnp.int32)[:, None] * N).reshape(M)

    ids = jnp.arange(M, dtype=jnp.float32)[:, None]
    combined = jnp.concatenate(
        [inputs.reshape(M, D), ids,
         jnp.zeros((M, _WROW - D - 1), jnp.float32)], axis=1)

    parts = _make_row_scatter(M, _WROW)(combined, pos_g)
    packed = _topk_blocks(parts.reshape(B, N, _WROW), B, nbins)
    rows = _make_row_gather(M, _WPACK)(packed.reshape(M, _WPACK), pos_g)

    rows = rows.reshape(B, N, _WPACK)
    dstg = rows[:, :, :_K].reshape(B, N * _K)
    valg = lax.bitcast_convert_type(rows[:, :, _K:2 * _K],
                                    jnp.float32).reshape(B, N * _K)

    bids = jnp.broadcast_to(
        jnp.arange(B, dtype=jnp.int32)[:, None], (B, N * _K))
    srcs = jnp.broadcast_to(
        jnp.repeat(jnp.arange(N, dtype=jnp.int32), _K)[None, :], (B, N * _K))
    full_idx = jnp.stack([bids, srcs, dstg], axis=-1)
    return full_idx, valg


# speedup vs baseline: 1.0524x; 1.0524x over previous
import functools

import jax
import jax.numpy as jnp
from jax import lax
from jax.experimental import pallas as pl
from jax.experimental.pallas import tpu as pltpu
from jax.experimental.pallas import tpu_sc as plsc

_BIN = 512
_K = 16
_DMULT = 0.1
_WROW = 384
_WPACK = 128
_CHUNK = 128


def _block_body(parts_ref, packed_ref):
    b = pl.program_id(0)
    blk = parts_ref[0]
    x = blk[:, :256]
    g = blk[:, 256].astype(jnp.int32) - b * 8192
    na = jnp.sum(x * x, axis=1)
    gram = lax.dot_general(x, x, (((1,), (1,)), ((), ())),
                           preferred_element_type=jnp.float32)
    dsq = na[:, None] - 2.0 * gram + na[None, :]
    cur = jnp.maximum(dsq, 1e-6)

    rowi = lax.broadcasted_iota(jnp.int32, (_BIN, _BIN), 0)
    comp_base = rowi * 8192 + g[:, None]
    big = jnp.int32(2 ** 30)

    dsts, vals = [], []
    for _ in range(_K):
        m = jnp.min(cur, axis=0)
        comp = jnp.where(cur == m[None, :], comp_base, big)
        r = jnp.min(comp, axis=0)
        selr = r // 8192
        dsts.append(r - selr * 8192)
        vals.append(m)
        cur = jnp.where(rowi == selr[None, :], jnp.float32(3e38), cur)

    dst = jnp.stack(dsts, axis=1)
    val = jnp.exp(-_DMULT * jnp.sqrt(jnp.stack(vals, axis=1)))

    rank = jnp.sum((dst[:, None, :] < dst[:, :, None]).astype(jnp.int32),
                   axis=2)
    sd, sv = [], []
    for u in range(_K):
        sel = rank == u
        sd.append(jnp.sum(jnp.where(sel, dst, 0), axis=1))
        sv.append(jnp.sum(jnp.where(sel, val, 0.0), axis=1))
    packed_ref[0] = jnp.concatenate(
        [jnp.stack(sd, axis=1),
         lax.bitcast_convert_type(jnp.stack(sv, axis=1), jnp.int32),
         jnp.zeros((_BIN, _WPACK - 2 * _K), jnp.int32)], axis=1)


def _topk_blocks(parts, B, nbins):
    return pl.pallas_call(
        _block_body,
        grid=(B, nbins),
        in_specs=[pl.BlockSpec((1, _BIN, _WROW), lambda b, n: (b, n, 0))],
        out_specs=pl.BlockSpec((1, _BIN, _WPACK), lambda b, n: (b, n, 0)),
        out_shape=jax.ShapeDtypeStruct((B, nbins * _BIN, _WPACK), jnp.int32),
        compiler_params=pltpu.CompilerParams(
            dimension_semantics=("parallel", "parallel")),
    )(parts)


def _make_row_scatter(M, W):
    info = plsc.get_sparse_core_info()
    NC, NS = info.num_cores, info.num_subcores
    per_w = M // (NC * NS)
    n_chunks = per_w // _CHUNK
    mesh = plsc.VectorSubcoreMesh(core_axis_name="c", subcore_axis_name="s")

    @functools.partial(
        pl.kernel, mesh=mesh,
        out_type=jax.ShapeDtypeStruct((M, W), jnp.float32),
        scratch_types=[pltpu.VMEM((_CHUNK,), jnp.int32),
                       pltpu.VMEM((_CHUNK, W), jnp.float32),
                       pltpu.SemaphoreType.DMA],
    )
    def sk(rows_hbm, idx_hbm, out_hbm, idx_v, rows_v, sem):
        wid = lax.axis_index("s") * NC + lax.axis_index("c")
        base = wid * per_w
        for c in range(n_chunks):
            off = base + c * _CHUNK
            pltpu.sync_copy(rows_hbm.at[pl.ds(off, _CHUNK)], rows_v)
            pltpu.sync_copy(idx_hbm.at[pl.ds(off, _CHUNK)], idx_v)
            pltpu.async_copy(rows_v, out_hbm.at[idx_v], sem).wait()

    return sk


def _make_row_gather(M, W):
    info = plsc.get_sparse_core_info()
    NC, NS = info.num_cores, info.num_subcores
    per_w = M // (NC * NS)
    n_chunks = per_w // _CHUNK
    mesh = plsc.VectorSubcoreMesh(core_axis_name="c", subcore_axis_name="s")

    @functools.partial(
        pl.kernel, mesh=mesh,
        out_type=jax.ShapeDtypeStruct((M, W), jnp.int32),
        scratch_types=[pltpu.VMEM((_CHUNK,), jnp.int32),
                       pltpu.VMEM((_CHUNK, W), jnp.int32),
                       pltpu.SemaphoreType.DMA],
    )
    def gk(table_hbm, idx_hbm, out_hbm, idx_v, rows_v, sem):
        wid = lax.axis_index("s") * NC + lax.axis_index("c")
        base = wid * per_w
        for c in range(n_chunks):
            off = base + c * _CHUNK
            pltpu.sync_copy(idx_hbm.at[pl.ds(off, _CHUNK)], idx_v)
            pltpu.async_copy(table_hbm.at[idx_v], rows_v, sem).wait()
            pltpu.sync_copy(rows_v, out_hbm.at[pl.ds(off, _CHUNK)])

    return gk


def kernel(inputs, codebook):
    B, N, D = inputs.shape
    nbins = N // _BIN
    M = B * N

    mul = inputs @ codebook[:, : nbins // 2]
    cmul = jnp.concatenate([mul, -mul], axis=-1)
    bin_idx = jnp.argmax(cmul, axis=-1).astype(jnp.int32)

    oh = (bin_idx[:, None, :] ==
          jnp.arange(nbins, dtype=jnp.int32)[None, :, None]).astype(jnp.int32)
    csum = jnp.cumsum(oh, axis=2)
    counts = csum[:, :, -1]
    starts = jnp.cumsum(counts, axis=1) - counts
    pos = (jnp.sum(oh * csum, axis=1) - 1
           + jnp.sum(oh * starts[:, :, None], axis=1))
    pos_g = (pos + jnp.arange(B, dtype=jnp.int32)[:, None] * N).reshape(M)

    ids = jnp.arange(M, dtype=jnp.float32)[:, None]
    combined = jnp.concatenate(
        [inputs.reshape(M, D), ids,
         jnp.zeros((M, _WROW - D - 1), jnp.float32)], axis=1)

    parts = _make_row_scatter(M, _WROW)(combined, pos_g)
    packed = _topk_blocks(parts.reshape(B, N, _WROW), B, nbins)
    rows = _make_row_gather(M, _WPACK)(packed.reshape(M, _WPACK), pos_g)

    rows = rows.reshape(B, N, _WPACK)
    dstg = rows[:, :, :_K].reshape(B, N * _K)
    valg = lax.bitcast_convert_type(rows[:, :, _K:2 * _K],
                                    jnp.float32).reshape(B, N * _K)

    bids = jnp.broadcast_to(
        jnp.arange(B, dtype=jnp.int32)[:, None], (B, N * _K))
    srcs = jnp.broadcast_to(
        jnp.repeat(jnp.arange(N, dtype=jnp.int32), _K)[None, :], (B, N * _K))
    full_idx = jnp.stack([bids, srcs, dstg], axis=-1)
    return full_idx, valg


# speedup vs baseline: 1.0552x; 1.0026x over previous
import functools

import jax
import jax.numpy as jnp
from jax import lax
from jax.experimental import pallas as pl
from jax.experimental.pallas import tpu as pltpu
from jax.experimental.pallas import tpu_sc as plsc

_BIN = 512
_K = 16
_DMULT = 0.1
_WROW = 384
_WPACK = 128
_CHUNK = 128


def _block_body(parts_ref, packed_ref):
    b = pl.program_id(0)
    blk = parts_ref[0]
    x = blk[:, :256]
    g = blk[:, 256].astype(jnp.int32) - b * 8192
    na = jnp.sum(x * x, axis=1)
    gram = lax.dot_general(x, x, (((1,), (1,)), ((), ())),
                           preferred_element_type=jnp.float32)
    dsq = na[:, None] - 2.0 * gram + na[None, :]
    cur = jnp.maximum(dsq, 1e-6)

    rowi = lax.broadcasted_iota(jnp.int32, (_BIN, _BIN), 0)
    comp_base = rowi * 8192 + g[:, None]
    big = jnp.int32(2 ** 30)

    dsts, vals = [], []
    for _ in range(_K):
        m = jnp.min(cur, axis=0)
        comp = jnp.where(cur == m[None, :], comp_base, big)
        r = jnp.min(comp, axis=0)
        selr = r // 8192
        dsts.append(r - selr * 8192)
        vals.append(m)
        cur = jnp.where(rowi == selr[None, :], jnp.float32(3e38), cur)

    dst = jnp.stack(dsts, axis=1)
    val = jnp.exp(-_DMULT * jnp.sqrt(jnp.stack(vals, axis=1)))

    rank = jnp.sum((dst[:, None, :] < dst[:, :, None]).astype(jnp.int32),
                   axis=2)
    sd, sv = [], []
    for u in range(_K):
        sel = rank == u
        sd.append(jnp.sum(jnp.where(sel, dst, 0), axis=1))
        sv.append(jnp.sum(jnp.where(sel, val, 0.0), axis=1))
    packed_ref[0] = jnp.concatenate(
        [jnp.stack(sd, axis=1),
         lax.bitcast_convert_type(jnp.stack(sv, axis=1), jnp.int32),
         jnp.zeros((_BIN, _WPACK - 2 * _K), jnp.int32)], axis=1)


def _topk_blocks(parts, B, nbins):
    return pl.pallas_call(
        _block_body,
        grid=(B, nbins),
        in_specs=[pl.BlockSpec((1, _BIN, _WROW), lambda b, n: (b, n, 0))],
        out_specs=pl.BlockSpec((1, _BIN, _WPACK), lambda b, n: (b, n, 0)),
        out_shape=jax.ShapeDtypeStruct((B, nbins * _BIN, _WPACK), jnp.int32),
        compiler_params=pltpu.CompilerParams(
            dimension_semantics=("parallel", "parallel")),
    )(parts)


def _make_row_scatter(M, W):
    info = plsc.get_sparse_core_info()
    NC, NS = info.num_cores, info.num_subcores
    per_w = M // (NC * NS)
    n_chunks = per_w // _CHUNK
    mesh = plsc.VectorSubcoreMesh(core_axis_name="c", subcore_axis_name="s")

    @functools.partial(
        pl.kernel, mesh=mesh,
        out_type=jax.ShapeDtypeStruct((M, W), jnp.float32),
        scratch_types=[pltpu.VMEM((_CHUNK,), jnp.int32),
                       pltpu.VMEM((_CHUNK, W), jnp.float32),
                       pltpu.SemaphoreType.DMA],
    )
    def sk(rows_hbm, idx_hbm, out_hbm, idx_v, rows_v, sem):
        wid = lax.axis_index("s") * NC + lax.axis_index("c")
        base = wid * per_w
        for c in range(n_chunks):
            off = base + c * _CHUNK
            pltpu.sync_copy(rows_hbm.at[pl.ds(off, _CHUNK)], rows_v)
            pltpu.sync_copy(idx_hbm.at[pl.ds(off, _CHUNK)], idx_v)
            pltpu.async_copy(rows_v, out_hbm.at[idx_v], sem).wait()

    return sk


def _make_row_gather(M, W):
    info = plsc.get_sparse_core_info()
    NC, NS = info.num_cores, info.num_subcores
    per_w = M // (NC * NS)
    n_chunks = per_w // _CHUNK
    mesh = plsc.VectorSubcoreMesh(core_axis_name="c", subcore_axis_name="s")

    @functools.partial(
        pl.kernel, mesh=mesh,
        out_type=jax.ShapeDtypeStruct((M, W), jnp.int32),
        scratch_types=[pltpu.VMEM((_CHUNK,), jnp.int32),
                       pltpu.VMEM((_CHUNK, W), jnp.int32),
                       pltpu.SemaphoreType.DMA],
    )
    def gk(table_hbm, idx_hbm, out_hbm, idx_v, rows_v, sem):
        wid = lax.axis_index("s") * NC + lax.axis_index("c")
        base = wid * per_w
        for c in range(n_chunks):
            off = base + c * _CHUNK
            pltpu.sync_copy(idx_hbm.at[pl.ds(off, _CHUNK)], idx_v)
            pltpu.async_copy(table_hbm.at[idx_v], rows_v, sem).wait()
            pltpu.sync_copy(rows_v, out_hbm.at[pl.ds(off, _CHUNK)])

    return gk


def kernel(inputs, codebook):
    B, N, D = inputs.shape
    nbins = N // _BIN
    M = B * N

    mul = inputs @ codebook[:, : nbins // 2]
    cmul = jnp.concatenate([mul, -mul], axis=-1)
    bin_idx = jnp.argmax(cmul, axis=-1).astype(jnp.int32)

    oh = (bin_idx[:, None, :] ==
          jnp.arange(nbins, dtype=jnp.int32)[None, :, None]
          ).astype(jnp.float32)
    nblk = N // 128
    i128 = jnp.arange(128)
    incl128 = (i128[:, None] <= i128[None, :]).astype(jnp.float32)
    iblk = jnp.arange(nblk)
    exclb = (iblk[:, None] < iblk[None, :]).astype(jnp.float32)
    ibin = jnp.arange(nbins)
    exclv = (ibin[:, None] < ibin[None, :]).astype(jnp.float32)

    o4 = oh.reshape(B * nbins * nblk, 128)
    c1 = o4 @ incl128
    rs = c1[:, -1].reshape(B * nbins, nblk)
    c2 = rs @ exclb
    full = (c2[:, :, None] + c1.reshape(B * nbins, nblk, 128)
            ).reshape(B, nbins, N)
    counts = (c2[:, -1] + rs[:, -1]).reshape(B, nbins)
    starts = counts @ exclv
    pos_f = jnp.sum(oh * (full + starts[:, :, None] - 1.0), axis=1)
    pos = pos_f.astype(jnp.int32)
    pos_g = (pos + jnp.arange(B, dtype=jnp.int32)[:, None] * N).reshape(M)

    ids = jnp.arange(M, dtype=jnp.float32)[:, None]
    combined = jnp.concatenate(
        [inputs.reshape(M, D), ids,
         jnp.zeros((M, _WROW - D - 1), jnp.float32)], axis=1)

    parts = _make_row_scatter(M, _WROW)(combined, pos_g)
    packed = _topk_blocks(parts.reshape(B, N, _WROW), B, nbins)
    rows = _make_row_gather(M, _WPACK)(packed.reshape(M, _WPACK), pos_g)

    rows = rows.reshape(B, N, _WPACK)
    dstg = rows[:, :, :_K].reshape(B, N * _K)
    valg = lax.bitcast_convert_type(rows[:, :, _K:2 * _K],
                                    jnp.float32).reshape(B, N * _K)

    bids = jnp.broadcast_to(
        jnp.arange(B, dtype=jnp.int32)[:, None], (B, N * _K))
    srcs = jnp.broadcast_to(
        jnp.repeat(jnp.arange(N, dtype=jnp.int32), _K)[None, :], (B, N * _K))
    full_idx = jnp.stack([bids, srcs, dstg], axis=-1)
    return full_idx, valg
